# Initial kernel scaffold; baseline (speedup 1.0000x reference)
import jax
import jax.numpy as jnp
from jax.experimental import pallas as pl


def kernel(x):
    raise NotImplementedError("write your pallas kernel here")


# speedup vs baseline: 33.1519x; 33.1519x over previous
import jax
import jax.numpy as jnp
from jax.experimental import pallas as pl

_R = 6
_C = 5
_H = 64
_W = 128
_BB = 8


def _pad_kernel(x_ref, o_ref):
    x = x_ref[...]

    vmask = (jnp.arange(_W) != 0) & (jnp.arange(_W) != _W // 2)
    row0 = x[:, :, :, 0, :] * vmask.astype(x.dtype)

    row_first_l = row0[:, :, :, 0:_W // 2]
    row_first_r = row0[:, :, :, _W // 2:]
    row_last_l = x[:, :, :, _H - 1, 0:_W // 2]
    row_last_r = x[:, :, :, _H - 1, _W // 2:]
    col_first = x[:, :, :, :, 0] * (jnp.arange(_H) != 0).astype(x.dtype)
    col_last = x[:, :, :, :, _W - 1]

    roll_c = lambda a, s: jnp.roll(a, s, axis=2)
    roll_r = lambda a, s: jnp.roll(a, s, axis=1)
    ii = jax.lax.broadcasted_iota(jnp.int32, (_H, _H), 0)
    jj = jax.lax.broadcasted_iota(jnp.int32, (_H, _H), 1)
    rev_p = (ii + jj == _H - 1).astype(x.dtype)
    flip = lambda a: jax.lax.dot_general(
        a, rev_p, (((3,), (0,)), ((), ())), preferred_element_type=jnp.float32)

    top_l = roll_c(row_last_r, 1)
    top_r = flip(roll_r(roll_c(col_last, 1), -1))
    bot_l = flip(roll_r(roll_c(col_first, -1), -1))
    bot_r = roll_c(row_first_l, -1)
    left = flip(roll_r(roll_c(row_last_l, 1), 1))
    right = flip(roll_r(roll_c(row_first_r, -1), 1))

    o_ref[:, :, :, 1:2, 1:_W + 1] = row0[:, :, :, None, :]
    o_ref[:, :, :, 2:_H + 1, 1:_W + 1] = x[:, :, :, 1:, :]

    o_ref[:, :, :, 0:1, 1:_W // 2 + 1] = top_l[:, :, :, None, :]
    o_ref[:, :, :, 0:1, _W // 2 + 1:_W + 1] = top_r[:, :, :, None, :]
    o_ref[:, :, :, _H + 1:_H + 2, 2:_W // 2 + 2] = bot_l[:, :, :, None, :]
    o_ref[:, :, :, _H + 1:_H + 2, _W // 2 + 1:_W + 1] = bot_r[:, :, :, None, :]
    o_ref[:, :, :, 1:_H + 1, 0:1] = left[:, :, :, :, None]
    o_ref[:, :, :, 2:_H + 2, _W + 1:_W + 2] = right[:, :, :, :, None]

    zc = jnp.zeros(x.shape[:3] + (1, 1), x.dtype)
    o_ref[:, :, :, 0:1, 0:1] = zc
    o_ref[:, :, :, 0:1, _W + 1:_W + 2] = zc
    o_ref[:, :, :, 1:2, _W + 1:_W + 2] = zc
    o_ref[:, :, :, _H + 1:_H + 2, 0:1] = zc
    o_ref[:, :, :, _H + 1:_H + 2, 1:2] = zc


def kernel(x):
    lead = x.shape[:2]
    b = lead[0] * lead[1]
    xf = x.reshape((b, _R, _C, _H, _W))
    out = pl.pallas_call(
        _pad_kernel,
        grid=(b // _BB,),
        in_specs=[pl.BlockSpec((_BB, _R, _C, _H, _W), lambda i: (i, 0, 0, 0, 0))],
        out_specs=pl.BlockSpec((_BB, _R, _C, _H + 2, _W + 2), lambda i: (i, 0, 0, 0, 0)),
        out_shape=jax.ShapeDtypeStruct((b, _R, _C, _H + 2, _W + 2), x.dtype),
    )(xf)
    return out.reshape(lead + (_R, _C, _H + 2, _W + 2))
